# baseline (speedup 1.0000x reference)
import functools

import jax
import jax.numpy as jnp
from jax import lax
from jax.experimental import pallas as pl
from jax.experimental.pallas import tpu as pltpu
from jax.experimental.pallas import tpu_sc as plsc

NUM_GROUPS = 1000000
DIM = 64
BATCH_SIZE = 16384
PAIR_W = 2 * DIM
NUM_PAIRS = NUM_GROUPS // 2

_info = plsc.get_sparse_core_info()
_NC, _NS = _info.num_cores, _info.num_subcores
_NW = _NC * _NS
_B_PER_W = BATCH_SIZE // _NW
_CHUNK = 128
_NCHUNK = _B_PER_W // _CHUNK
_LANES = 16

_mesh = plsc.VectorSubcoreMesh(core_axis_name="c", subcore_axis_name="s")


@functools.partial(
    pl.kernel,
    mesh=_mesh,
    out_type=jax.ShapeDtypeStruct((BATCH_SIZE // 2, PAIR_W), jnp.float32),
    scratch_types=[
        pltpu.VMEM((_B_PER_W,), jnp.int32),
        pltpu.VMEM((_B_PER_W,), jnp.int32),
        pltpu.VMEM((_CHUNK, PAIR_W), jnp.float32),
        pltpu.VMEM((_B_PER_W // 2, PAIR_W), jnp.float32),
        pltpu.SemaphoreType.DMA,
    ],
)
def _gather_kernel(idx_hbm, table_hbm, out_hbm, idx_v, pid_v, pairs_v, rows_v,
                   sem):
    wid = lax.axis_index("s") * _NC + lax.axis_index("c")
    base = wid * _B_PER_W
    pltpu.sync_copy(idx_hbm.at[pl.ds(base, _B_PER_W)], idx_v)
    for i in range(_B_PER_W // _LANES):
        sl = pl.ds(i * _LANES, _LANES)
        pid_v[sl] = lax.shift_right_logical(idx_v[sl], 1)
    for j in range(_NCHUNK):
        pltpu.async_copy(
            table_hbm.at[pid_v.at[pl.ds(j * _CHUNK, _CHUNK)]],
            pairs_v,
            sem,
        ).wait()
        for g in range(_CHUNK // _LANES):
            offv = lax.bitwise_and(
                idx_v[pl.ds(j * _CHUNK + g * _LANES, _LANES)], jnp.int32(1)
            ) * jnp.int32(DIM)
            for k16 in range(_LANES):
                k = g * _LANES + k16
                b = j * _CHUNK + k
                off = offv[k16]
                for c in range(DIM // _LANES):
                    dst = pl.ds((b % 2) * DIM + c * _LANES, _LANES)
                    rows_v[b // 2, dst] = pairs_v[k, pl.ds(off + c * _LANES,
                                                           _LANES)]
    out_off = pl.multiple_of(base // 2, _B_PER_W // 2)
    pltpu.sync_copy(rows_v, out_hbm.at[pl.ds(out_off, _B_PER_W // 2)])


def kernel(num_group, table):
    idx = num_group.astype(jnp.int32)
    pairs = table.reshape(NUM_PAIRS, PAIR_W)
    out_pairs = _gather_kernel(idx, pairs)
    return out_pairs.reshape(BATCH_SIZE, DIM)


# speedup vs baseline: 1.1510x; 1.1510x over previous
import functools

import jax
import jax.numpy as jnp
from jax import lax
from jax.experimental import pallas as pl
from jax.experimental.pallas import tpu as pltpu
from jax.experimental.pallas import tpu_sc as plsc

NUM_GROUPS = 1000000
DIM = 64
BATCH_SIZE = 16384
PAD_W = 128

_info = plsc.get_sparse_core_info()
_NC, _NS = _info.num_cores, _info.num_subcores
_NW = _NC * _NS
_B_PER_W = BATCH_SIZE // _NW
_CHUNK = 128
_NCHUNK = _B_PER_W // _CHUNK

_mesh = plsc.VectorSubcoreMesh(core_axis_name="c", subcore_axis_name="s")


@functools.partial(
    pl.kernel,
    mesh=_mesh,
    out_type=jax.ShapeDtypeStruct((BATCH_SIZE, PAD_W), jnp.float32),
    scratch_types=[
        pltpu.VMEM((_B_PER_W,), jnp.int32),
        pltpu.VMEM((_B_PER_W, PAD_W), jnp.float32),
        pltpu.SemaphoreType.DMA,
    ],
)
def _gather_kernel(idx_hbm, table_hbm, out_hbm, idx_v, rows_v, sem):
    wid = lax.axis_index("s") * _NC + lax.axis_index("c")
    base = pl.multiple_of(wid * _B_PER_W, _B_PER_W)
    pltpu.sync_copy(idx_hbm.at[pl.ds(base, _B_PER_W)], idx_v)
    copies = []
    for j in range(_NCHUNK):
        copies.append(
            pltpu.async_copy(
                table_hbm.at[idx_v.at[pl.ds(j * _CHUNK, _CHUNK)]],
                rows_v.at[pl.ds(j * _CHUNK, _CHUNK)],
                sem,
            )
        )
    for c in copies:
        c.wait()
    pltpu.sync_copy(rows_v, out_hbm.at[pl.ds(base, _B_PER_W)])


def kernel(num_group, table):
    idx = num_group.astype(jnp.int32)
    padded = jnp.pad(table, ((0, 0), (0, PAD_W - DIM)))
    return _gather_kernel(idx, padded)[:, :DIM]


# speedup vs baseline: 1.5802x; 1.3730x over previous
import functools

import jax
import jax.numpy as jnp
from jax import lax
from jax.experimental import pallas as pl
from jax.experimental.pallas import tpu as pltpu
from jax.experimental.pallas import tpu_sc as plsc

NUM_GROUPS = 1000000
DIM = 64
BATCH_SIZE = 16384

VRANGE = 32768
SBW = 512
TAIL_C0 = 999936
TAIL_W = NUM_GROUPS - TAIL_C0
OUT_ROWS = BATCH_SIZE + 128
DUMMY = BATCH_SIZE
BATCH_ROWS = 128

_info = plsc.get_sparse_core_info()
_NC, _NS = _info.num_cores, _info.num_subcores
_NW = _NC * _NS
_LANES = 16

_mesh = plsc.VectorSubcoreMesh(core_axis_name="c", subcore_axis_name="s")


@functools.partial(
    pl.kernel,
    mesh=_mesh,
    out_type=jax.ShapeDtypeStruct((OUT_ROWS, 2 * DIM), jnp.float32),
    scratch_types=[
        pltpu.VMEM((BATCH_SIZE,), jnp.int32),
        pltpu.VMEM((BATCH_SIZE + _LANES,), jnp.int32),
        pltpu.VMEM((BATCH_SIZE + _LANES,), jnp.int32),
        pltpu.VMEM((DIM, SBW), jnp.float32),
        pltpu.VMEM((DIM, TAIL_W), jnp.float32),
        pltpu.VMEM((BATCH_ROWS, 2 * DIM), jnp.float32),
        pltpu.VMEM((1, BATCH_ROWS), jnp.int32),
        pltpu.SMEM((2,), jnp.int32),
        pltpu.SemaphoreType.DMA,
    ],
    compiler_params=pltpu.CompilerParams(needs_layout_passes=False),
)
def _stream_gather(idx_hbm, tt_hbm, ttail_hbm, out_hbm, idx_all, my_idx, my_b,
                   staged, tail_v, rows_buf, b_batch, cnt_s, sem):
    wid = lax.axis_index("s") * _NC + lax.axis_index("c")
    lane = lax.iota(jnp.int32, _LANES)
    pltpu.sync_copy(idx_hbm, idx_all)

    def scan_body(i, cnt):
        v = idx_all[pl.ds(i * _LANES, _LANES)]
        m = lax.shift_right_logical(v, 15) == wid
        plsc.store_compressed(my_idx.at[pl.ds(cnt, _LANES)], v, mask=m)
        plsc.store_compressed(my_b.at[pl.ds(cnt, _LANES)], i * _LANES + lane,
                              mask=m)
        return cnt + plsc.all_reduce_population_count(m)[0]

    nh = lax.fori_loop(0, BATCH_SIZE // _LANES, scan_body, jnp.int32(0))
    cnt_s[0] = nh
    cnt_s[1] = jnp.int32(0)

    def flush():
        bc = cnt_s[1]
        for k in range(BATCH_ROWS // _LANES):
            sl = pl.ds(k * _LANES, _LANES)
            pos = k * _LANES + lane
            b_batch[0, sl] = jnp.where(pos >= bc, jnp.int32(DUMMY),
                                       b_batch[0, sl])
        pltpu.async_copy(rows_buf, out_hbm.at[b_batch.at[0]], sem).wait()
        cnt_s[1] = jnp.int32(0)

    def emit_block(c0, width, src):
        n_hits = cnt_s[0]

        def grp_body(g, _):
            vi = my_idx[pl.ds(g * _LANES, _LANES)]
            vb = my_b[pl.ds(g * _LANES, _LANES)]
            valid = (g * _LANES + lane) < n_hits
            m = valid & (vi >= c0) & (vi < c0 + width)
            npc = plsc.all_reduce_population_count(m)[0]

            @pl.when(npc > 0)
            def _():
                @pl.when(cnt_s[1] > BATCH_ROWS - _LANES)
                def _():
                    flush()

                bc = cnt_s[1]
                slots = bc + plsc.cumsum(m.astype(jnp.int32)) - 1
                col = vi - c0
                plsc.store_scatter(b_batch, [jnp.zeros((_LANES,), jnp.int32),
                                             slots], vb, mask=m)
                for d in range(DIM):
                    dv = jnp.full((_LANES,), d, jnp.int32)
                    val = plsc.load_gather(src, [dv, col], mask=m)
                    plsc.store_scatter(rows_buf, [slots, dv], val, mask=m)
                cnt_s[1] = bc + npc

            return 0

        lax.fori_loop(0, (n_hits + _LANES - 1) // _LANES, grp_body, 0)

    base_c = wid * VRANGE
    nfull = jnp.where(wid < 30, VRANGE // SBW,
                      jnp.where(wid == 30, (TAIL_C0 - 30 * VRANGE) // SBW, 0))

    def block_body(s, _):
        c0 = pl.multiple_of(base_c + s * SBW, SBW)
        pltpu.sync_copy(tt_hbm.at[:, pl.ds(c0, SBW)], staged)
        emit_block(c0, SBW, staged)
        return 0

    lax.fori_loop(0, nfull, block_body, 0)

    @pl.when(wid == _NW - 2)
    def _():
        pltpu.sync_copy(ttail_hbm, tail_v)
        emit_block(jnp.int32(TAIL_C0), TAIL_W, tail_v)

    flush()


def kernel(num_group, table):
    idx = num_group.astype(jnp.int32)
    ttail = table[TAIL_C0:, :].T
    out = _stream_gather(idx, table.T, ttail)
    return out[:BATCH_SIZE, :DIM]
